# baseline (speedup 1.0000x reference)
import jax
import jax.numpy as jnp
from jax.experimental import pallas as pl
from jax.experimental.pallas import tpu as pltpu

_WAYS = 2


def _make_kernel(bi, sub):
    def _fused(*refs):
        adj_refs = refs[:_WAYS]
        (x_ref, w1_ref, b1_ref, w2_ref, b2_ref, we_ref, be_ref, att_ref,
         out_ref, y_ref, s1_scr, s2_scr) = refs[_WAYS:]
        p = pl.program_id(0)
        i = pl.program_id(1)

        @pl.when((p == 0) & (i == 0))
        def _():
            s1_scr[...] = jnp.dot(x_ref[...], w1_ref[...],
                                  preferred_element_type=jnp.float32)

        @pl.when(p == 0)
        def _():
            for j in range(_WAYS):
                loc = pl.ds(j * sub, sub)
                acc = jnp.dot(adj_refs[j][...], s1_scr[...],
                              preferred_element_type=jnp.float32)
                h = jnp.maximum(acc + b1_ref[...], 0.0)
                s2_scr[pl.ds(i * bi + j * sub, sub), :] = jnp.dot(
                    h, w2_ref[...], preferred_element_type=jnp.float32)
                g = jax.lax.dot_general(h, we_ref[...],
                                        (((1,), (1,)), ((), ())),
                                        preferred_element_type=jnp.float32)
                y_ref[loc, :] = (jax.nn.sigmoid(g + be_ref[...])
                                 * att_ref[loc, :])

        @pl.when(p == 1)
        def _():
            for j in range(_WAYS):
                loc = pl.ds(j * sub, sub)
                o = jnp.dot(adj_refs[j][...], s2_scr[...],
                            preferred_element_type=jnp.float32) + b2_ref[...]
                m = jnp.max(o, axis=1, keepdims=True)
                lse = jnp.log(jnp.sum(jnp.exp(o - m), axis=1, keepdims=True))
                out_ref[loc, :] = o - m - lse

    return _fused


def _adj_spec(n, sub, j):
    return pl.BlockSpec((sub, n), lambda p, i, j=j: (_WAYS * i + j, 0))


def kernel(x, adj, att, W1, b1, W2, b2, We, be):
    n, nfeat = x.shape
    nhid = W1.shape[1]
    nclass = W2.shape[1]
    nstruc = We.shape[0]

    bi = 400 if n % 400 == 0 else n
    ni = n // bi
    sub = bi // _WAYS

    y_map = lambda p, i: (i * (1 - p) + (ni - 1) * p, 0)
    out_map = lambda p, i: (i * p, 0)

    out, y = pl.pallas_call(
        _make_kernel(bi, sub),
        grid=(2, ni),
        in_specs=[_adj_spec(n, sub, j) for j in range(_WAYS)] + [
            pl.BlockSpec((n, nfeat), lambda p, i: (0, 0)),
            pl.BlockSpec((nfeat, nhid), lambda p, i: (0, 0)),
            pl.BlockSpec((1, nhid), lambda p, i: (0, 0)),
            pl.BlockSpec((nhid, nclass), lambda p, i: (0, 0)),
            pl.BlockSpec((1, nclass), lambda p, i: (0, 0)),
            pl.BlockSpec((nstruc, nhid), lambda p, i: (0, 0)),
            pl.BlockSpec((1, nstruc), lambda p, i: (0, 0)),
            pl.BlockSpec((bi, nstruc), y_map),
        ],
        out_specs=(
            pl.BlockSpec((bi, nclass), out_map),
            pl.BlockSpec((bi, nstruc), y_map),
        ),
        out_shape=(
            jax.ShapeDtypeStruct((n, nclass), jnp.float32),
            jax.ShapeDtypeStruct((n, nstruc), jnp.float32),
        ),
        scratch_shapes=[
            pltpu.VMEM((n, nhid), jnp.float32),
            pltpu.VMEM((n, nclass), jnp.float32),
        ],
        compiler_params=pltpu.CompilerParams(
            dimension_semantics=("arbitrary", "arbitrary")),
    )(*([adj] * _WAYS), x, W1, b1.reshape(1, nhid), W2,
      b2.reshape(1, nclass), We, be.reshape(1, nstruc), att)

    return out, y


# speedup vs baseline: 1.0302x; 1.0302x over previous
import jax
import jax.numpy as jnp
from jax.experimental import pallas as pl
from jax.experimental.pallas import tpu as pltpu

_WAYS = 5


def _make_kernel(bi, sub):
    def _fused(*refs):
        adj_refs = refs[:_WAYS]
        (x_ref, w1_ref, b1_ref, w2_ref, b2_ref, we_ref, be_ref, att_ref,
         out_ref, y_ref, s1_scr, h_scr, s2_scr) = refs[_WAYS:]
        p = pl.program_id(0)
        i = pl.program_id(1)

        @pl.when((p == 0) & (i == 0))
        def _():
            s1_scr[...] = jnp.dot(x_ref[...], w1_ref[...],
                                  preferred_element_type=jnp.float32)

        @pl.when(p == 0)
        def _():
            for j in range(_WAYS):
                acc = jnp.dot(adj_refs[j][...], s1_scr[...],
                              preferred_element_type=jnp.float32)
                h_scr[pl.ds(i * bi + j * sub, sub), :] = (
                    jnp.maximum(acc + b1_ref[...], 0.0))

        @pl.when((p == 1) & (i == 0))
        def _():
            s2_scr[...] = jnp.dot(h_scr[...], w2_ref[...],
                                  preferred_element_type=jnp.float32)

        @pl.when(p == 1)
        def _():
            for j in range(_WAYS):
                loc = pl.ds(j * sub, sub)
                o = jnp.dot(adj_refs[j][...], s2_scr[...],
                            preferred_element_type=jnp.float32) + b2_ref[...]
                m = jnp.max(o, axis=1, keepdims=True)
                lse = jnp.log(jnp.sum(jnp.exp(o - m), axis=1, keepdims=True))
                out_ref[loc, :] = o - m - lse
                hh = h_scr[pl.ds(i * bi + j * sub, sub), :]
                g = jax.lax.dot_general(hh, we_ref[...],
                                        (((1,), (1,)), ((), ())),
                                        preferred_element_type=jnp.float32)
                y_ref[loc, :] = (jax.nn.sigmoid(g + be_ref[...])
                                 * att_ref[loc, :])

    return _fused


def _adj_spec(n, sub, j):
    return pl.BlockSpec((sub, n), lambda p, i, j=j: (_WAYS * i + j, 0))


def kernel(x, adj, att, W1, b1, W2, b2, We, be):
    n, nfeat = x.shape
    nhid = W1.shape[1]
    nclass = W2.shape[1]
    nstruc = We.shape[0]

    bi = 400 if n % 400 == 0 else n
    ni = n // bi
    sub = bi // _WAYS

    out, y = pl.pallas_call(
        _make_kernel(bi, sub),
        grid=(2, ni),
        in_specs=[_adj_spec(n, sub, j) for j in range(_WAYS)] + [
            pl.BlockSpec((n, nfeat), lambda p, i: (0, 0)),
            pl.BlockSpec((nfeat, nhid), lambda p, i: (0, 0)),
            pl.BlockSpec((1, nhid), lambda p, i: (0, 0)),
            pl.BlockSpec((nhid, nclass), lambda p, i: (0, 0)),
            pl.BlockSpec((1, nclass), lambda p, i: (0, 0)),
            pl.BlockSpec((nstruc, nhid), lambda p, i: (0, 0)),
            pl.BlockSpec((1, nstruc), lambda p, i: (0, 0)),
            pl.BlockSpec((bi, nstruc), lambda p, i: (i * p, 0)),
        ],
        out_specs=(
            pl.BlockSpec((bi, nclass), lambda p, i: (i * p, 0)),
            pl.BlockSpec((bi, nstruc), lambda p, i: (i * p, 0)),
        ),
        out_shape=(
            jax.ShapeDtypeStruct((n, nclass), jnp.float32),
            jax.ShapeDtypeStruct((n, nstruc), jnp.float32),
        ),
        scratch_shapes=[
            pltpu.VMEM((n, nhid), jnp.float32),
            pltpu.VMEM((n, nhid), jnp.float32),
            pltpu.VMEM((n, nclass), jnp.float32),
        ],
        compiler_params=pltpu.CompilerParams(
            dimension_semantics=("arbitrary", "arbitrary")),
    )(*([adj] * _WAYS), x, W1, b1.reshape(1, nhid), W2,
      b2.reshape(1, nclass), We, be.reshape(1, nstruc), att)

    return out, y


# speedup vs baseline: 1.0467x; 1.0160x over previous
import jax
import jax.numpy as jnp
from jax.experimental import pallas as pl
from jax.experimental.pallas import tpu as pltpu

_WAYS = 2


def _make_kernel(bi, sub):
    def _fused(*refs):
        adj_refs = refs[:_WAYS]
        (x_ref, w1_ref, b1_ref, w2_ref, b2_ref, we_ref, be_ref, att_ref,
         out_ref, y_ref, s1_scr, h_scr, s2_scr) = refs[_WAYS:]
        p = pl.program_id(0)
        i = pl.program_id(1)

        @pl.when((p == 0) & (i == 0))
        def _():
            s1_scr[...] = jnp.dot(x_ref[...], w1_ref[...],
                                  preferred_element_type=jnp.float32)

        @pl.when(p == 0)
        def _():
            for j in range(_WAYS):
                acc = jnp.dot(adj_refs[j][...], s1_scr[...],
                              preferred_element_type=jnp.float32)
                h_scr[pl.ds(i * bi + j * sub, sub), :] = (
                    jnp.maximum(acc + b1_ref[...], 0.0))

        @pl.when((p == 1) & (i == 0))
        def _():
            s2_scr[...] = jnp.dot(h_scr[...], w2_ref[...],
                                  preferred_element_type=jnp.float32)

        @pl.when(p == 1)
        def _():
            for j in range(_WAYS):
                loc = pl.ds(j * sub, sub)
                o = jnp.dot(adj_refs[j][...], s2_scr[...],
                            preferred_element_type=jnp.float32) + b2_ref[...]
                m = jnp.max(o, axis=1, keepdims=True)
                lse = jnp.log(jnp.sum(jnp.exp(o - m), axis=1, keepdims=True))
                out_ref[loc, :] = o - m - lse
                hh = h_scr[pl.ds(i * bi + j * sub, sub), :]
                g = jax.lax.dot_general(hh, we_ref[...],
                                        (((1,), (1,)), ((), ())),
                                        preferred_element_type=jnp.float32)
                y_ref[loc, :] = (jax.nn.sigmoid(g + be_ref[...])
                                 * att_ref[loc, :])

    return _fused


def _adj_spec(n, sub, j):
    return pl.BlockSpec((sub, n), lambda p, i, j=j: (_WAYS * i + j, 0))


def kernel(x, adj, att, W1, b1, W2, b2, We, be):
    n, nfeat = x.shape
    nhid = W1.shape[1]
    nclass = W2.shape[1]
    nstruc = We.shape[0]

    bi = 400 if n % 400 == 0 else n
    ni = n // bi
    sub = bi // _WAYS

    out, y = pl.pallas_call(
        _make_kernel(bi, sub),
        grid=(2, ni),
        in_specs=[_adj_spec(n, sub, j) for j in range(_WAYS)] + [
            pl.BlockSpec((n, nfeat), lambda p, i: (0, 0)),
            pl.BlockSpec((nfeat, nhid), lambda p, i: (0, 0)),
            pl.BlockSpec((1, nhid), lambda p, i: (0, 0)),
            pl.BlockSpec((nhid, nclass), lambda p, i: (0, 0)),
            pl.BlockSpec((1, nclass), lambda p, i: (0, 0)),
            pl.BlockSpec((nstruc, nhid), lambda p, i: (0, 0)),
            pl.BlockSpec((1, nstruc), lambda p, i: (0, 0)),
            pl.BlockSpec((bi, nstruc), lambda p, i: (i * p, 0)),
        ],
        out_specs=(
            pl.BlockSpec((bi, nclass), lambda p, i: (i * p, 0)),
            pl.BlockSpec((bi, nstruc), lambda p, i: (i * p, 0)),
        ),
        out_shape=(
            jax.ShapeDtypeStruct((n, nclass), jnp.float32),
            jax.ShapeDtypeStruct((n, nstruc), jnp.float32),
        ),
        scratch_shapes=[
            pltpu.VMEM((n, nhid), jnp.float32),
            pltpu.VMEM((n, nhid), jnp.float32),
            pltpu.VMEM((n, nclass), jnp.float32),
        ],
        compiler_params=pltpu.CompilerParams(
            dimension_semantics=("arbitrary", "arbitrary")),
    )(*([adj] * _WAYS), x, W1, b1.reshape(1, nhid), W2,
      b2.reshape(1, nclass), We, be.reshape(1, nstruc), att)

    return out, y
